# baseline (speedup 1.0000x reference)
import jax
import jax.numpy as jnp
from jax.experimental import pallas as pl
from jax.experimental.pallas import tpu as pltpu

DIM = 768
CODEBOOK_SIZE = 1024
CODEBOOK_DIM = 32
EPS = 1e-5

BLK = 2048
NCHUNK = 2


def _rpq_kernel(x_ref, w_ref, cb_ref, out_ref):
    w = w_ref[...]
    cb = cb_ref[...]
    cn = jnp.sum(cb * cb, axis=1, keepdims=True)
    b_aug = jnp.concatenate([-2.0 * cb, cn], axis=1)
    bh = b_aug.astype(jnp.bfloat16)
    bl = (b_aug - bh.astype(jnp.float32)).astype(jnp.bfloat16)
    b_big = jnp.concatenate([bh, bh, bl], axis=1)

    c = BLK // NCHUNK
    for k in range(NCHUNK):
        xs = x_ref[pl.ds(k * c, c), :]
        proj = jax.lax.dot_general(
            xs, w,
            dimension_numbers=(((1,), (1,)), ((), ())),
            preferred_element_type=jnp.float32,
            precision=jax.lax.Precision.DEFAULT,
        )
        mean = jnp.mean(proj, axis=-1, keepdims=True)
        var = jnp.mean((proj - mean) ** 2, axis=-1, keepdims=True)
        p = (proj - mean) / jnp.sqrt(var + EPS)
        a_aug = jnp.concatenate([p, jnp.ones((c, 1), jnp.float32)], axis=1)
        ah = a_aug.astype(jnp.bfloat16)
        al = (a_aug - ah.astype(jnp.float32)).astype(jnp.bfloat16)
        a_big = jnp.concatenate([ah, al, ah], axis=1)
        d = jax.lax.dot_general(
            b_big, a_big,
            dimension_numbers=(((1,), (1,)), ((), ())),
            preferred_element_type=jnp.float32,
            precision=jax.lax.Precision.DEFAULT,
        )
        out_ref[pl.ds(k * c, c)] = jnp.argmin(d, axis=0).astype(jnp.int32)


@jax.jit
def kernel(x, W, codebook):
    B, L, _ = x.shape
    n_tok = B * L
    xf = x.reshape(n_tok, DIM)
    grid = n_tok // BLK
    codes = pl.pallas_call(
        _rpq_kernel,
        grid=(grid,),
        in_specs=[
            pl.BlockSpec((BLK, DIM), lambda i: (i, 0)),
            pl.BlockSpec((CODEBOOK_DIM, DIM), lambda i: (0, 0)),
            pl.BlockSpec((CODEBOOK_SIZE, CODEBOOK_DIM), lambda i: (0, 0)),
        ],
        out_specs=pl.BlockSpec((BLK,), lambda i: (i,)),
        out_shape=jax.ShapeDtypeStruct((n_tok,), jnp.int32),
        compiler_params=pltpu.CompilerParams(
            dimension_semantics=("parallel",),
        ),
    )(xf, W, codebook)
    return codes.reshape(B, L)


# speedup vs baseline: 1.1241x; 1.1241x over previous
import jax
import jax.numpy as jnp
from jax.experimental import pallas as pl
from jax.experimental.pallas import tpu as pltpu

DIM = 768
CODEBOOK_SIZE = 1024
CODEBOOK_DIM = 32
EPS = 1e-5

BLK = 2048
NCHUNK = 1


def _rpq_kernel(x_ref, w_ref, cb_ref, out_ref):
    w = w_ref[...]
    cb = cb_ref[...]
    cn = jnp.sum(cb * cb, axis=1, keepdims=True)
    b_aug = jnp.concatenate([-2.0 * cb, cn], axis=1)
    bh = b_aug.astype(jnp.bfloat16)
    bl = (b_aug - bh.astype(jnp.float32)).astype(jnp.bfloat16)
    b_big = jnp.concatenate([bh, bh, bl], axis=1)

    c = BLK // NCHUNK
    for k in range(NCHUNK):
        xs = x_ref[pl.ds(k * c, c), :]
        proj = jax.lax.dot_general(
            xs, w,
            dimension_numbers=(((1,), (1,)), ((), ())),
            preferred_element_type=jnp.float32,
            precision=jax.lax.Precision.DEFAULT,
        )
        mean = jnp.mean(proj, axis=-1, keepdims=True)
        meansq = jnp.mean(proj * proj, axis=-1, keepdims=True)
        var = meansq - mean * mean
        p = (proj - mean) / jnp.sqrt(var + EPS)
        a_aug = jnp.concatenate([p, jnp.ones((c, 1), jnp.float32)], axis=1)
        ah = a_aug.astype(jnp.bfloat16)
        al = (a_aug - ah.astype(jnp.float32)).astype(jnp.bfloat16)
        a_big = jnp.concatenate([ah, al, ah], axis=1)
        d = jax.lax.dot_general(
            b_big, a_big,
            dimension_numbers=(((1,), (1,)), ((), ())),
            preferred_element_type=jnp.float32,
            precision=jax.lax.Precision.DEFAULT,
        )
        out_ref[pl.ds(k * c, c)] = jnp.argmin(d, axis=0).astype(jnp.int32)


@jax.jit
def kernel(x, W, codebook):
    B, L, _ = x.shape
    n_tok = B * L
    xf = x.reshape(n_tok, DIM)
    grid = n_tok // BLK
    codes = pl.pallas_call(
        _rpq_kernel,
        grid=(grid,),
        in_specs=[
            pl.BlockSpec((BLK, DIM), lambda i: (i, 0)),
            pl.BlockSpec((CODEBOOK_DIM, DIM), lambda i: (0, 0)),
            pl.BlockSpec((CODEBOOK_SIZE, CODEBOOK_DIM), lambda i: (0, 0)),
        ],
        out_specs=pl.BlockSpec((BLK,), lambda i: (i,)),
        out_shape=jax.ShapeDtypeStruct((n_tok,), jnp.int32),
        compiler_params=pltpu.CompilerParams(
            dimension_semantics=("parallel",),
        ),
    )(xf, W, codebook)
    return codes.reshape(B, L)
